# baseline (speedup 1.0000x reference)
import jax
import jax.numpy as jnp
from jax import lax
from jax.experimental import pallas as pl
from jax.experimental.pallas import tpu as pltpu
from jax.experimental.pallas import tpu_sc as plsc

_P = 8
_C = 32
_G = 128
_SUB = 2
_CHUNK = _G * _SUB
_NS = 4
_NW = 32
_L = 16


def _sc_body(edges, table, ratios, *refs):
    idx_i = list(refs[0:_NS])
    idx_j = list(refs[_NS:2 * _NS])
    rows_i = list(refs[2 * _NS:3 * _NS])
    rows_j = list(refs[3 * _NS:4 * _NS])
    out = list(refs[4 * _NS:5 * _NS])
    sem_idx = list(refs[5 * _NS:6 * _NS])
    sem_g = list(refs[6 * _NS:7 * _NS])
    sem_o = list(refs[7 * _NS:8 * _NS])

    total = edges.shape[1]
    wid = lax.axis_index("s") * 2 + lax.axis_index("c")
    iota = lax.broadcasted_iota(jnp.int32, (_L,), 0)

    kmax = (total + _NW - 1) // _NW
    kmax += (-kmax) % _NS
    nsteps = kmax // _NS

    def fire_idx(b, c):
        pltpu.async_copy(edges.at[0, c], idx_i[b], sem_idx[b])
        pltpu.async_copy(edges.at[1, c], idx_j[b], sem_idx[b])

    def wait_idx(b, c):
        pltpu.make_async_copy(edges.at[0, c], idx_i[b], sem_idx[b]).wait()
        pltpu.make_async_copy(edges.at[1, c], idx_j[b], sem_idx[b]).wait()

    def fire_gathers(b):
        for r in range(_SUB):
            dst_i = rows_i[b].at[pl.ds(r * _G, _G), :]
            dst_j = rows_j[b].at[pl.ds(r * _G, _G), :]
            pltpu.async_copy(table.at[idx_i[b].at[r]], dst_i, sem_g[b])
            pltpu.async_copy(table.at[idx_j[b].at[r]], dst_j, sem_g[b])

    def wait_gathers(b):
        for r in range(_SUB):
            dst_i = rows_i[b].at[pl.ds(r * _G, _G), :]
            dst_j = rows_j[b].at[pl.ds(r * _G, _G), :]
            pltpu.make_async_copy(table.at[idx_i[b].at[r]], dst_i,
                                  sem_g[b]).wait()
            pltpu.make_async_copy(table.at[idx_j[b].at[r]], dst_j,
                                  sem_g[b]).wait()

    def fire_out(b, c):
        pltpu.async_copy(out[b],
                         ratios.at[pl.ds(c * _CHUNK, _CHUNK), :],
                         sem_o[b])

    def wait_out(b, c):
        pltpu.make_async_copy(out[b],
                              ratios.at[pl.ds(c * _CHUNK, _CHUNK), :],
                              sem_o[b]).wait()

    def compute(b):
        def g_body(g, carry):
            e0 = g * _L + iota
            for p in range(_P):
                c0 = jnp.full((_L,), 4 * p, jnp.int32)
                c1 = c0 + 1
                c2 = c0 + 2
                c3 = c0 + 3
                pw = plsc.load_gather(rows_i[b], [e0, c0])
                px = plsc.load_gather(rows_i[b], [e0, c1])
                py = plsc.load_gather(rows_i[b], [e0, c2])
                pz = plsc.load_gather(rows_i[b], [e0, c3])
                qw = plsc.load_gather(rows_j[b], [e0, c0])
                qx = plsc.load_gather(rows_j[b], [e0, c1])
                qy = plsc.load_gather(rows_j[b], [e0, c2])
                qz = plsc.load_gather(rows_j[b], [e0, c3])
                r = 1.0 / (qw * qw + qx * qx + qy * qy + qz * qz)
                ow = (pw * qw + px * qx + py * qy + pz * qz) * r
                ox = (px * qw - pw * qx - py * qz + pz * qy) * r
                oy = (py * qw - pw * qy + px * qz - pz * qx) * r
                oz = (pz * qw - pw * qz - px * qy + py * qx) * r
                plsc.store_scatter(out[b], [e0, c0], ow)
                plsc.store_scatter(out[b], [e0, c1], ox)
                plsc.store_scatter(out[b], [e0, c2], oy)
                plsc.store_scatter(out[b], [e0, c3], oz)
            return carry

        lax.fori_loop(0, _CHUNK // _L, g_body, 0)

    for b in range(_NS):
        fire_idx(b, wid + b * _NW)
    for b in range(_NS):
        wait_idx(b, wid + b * _NW)
        fire_gathers(b)

    def step(t, carry):
        for b in range(_NS):
            k = _NS * t + b
            c = wid + k * _NW
            cn = c + _NS * _NW

            def process():
                wait_gathers(b)

                @pl.when(cn < total)
                def _():
                    fire_idx(b, cn)

                @pl.when(t >= 1)
                def _():
                    wait_out(b, c - _NS * _NW)

                compute(b)
                fire_out(b, c)

                @pl.when(cn < total)
                def _():
                    wait_idx(b, cn)
                    fire_gathers(b)

            pl.when(c < total)(process)
        return carry

    lax.fori_loop(0, nsteps, step, 0)

    for b in range(_NS):
        c_last = wid + (kmax - _NS + b) * _NW

        @pl.when(c_last < total)
        def _():
            wait_out(b, c_last)


_WB = 3200


def _w_body(w_ref, o_ref):
    o_ref[...] = jnp.broadcast_to(w_ref[...], o_ref.shape)


def kernel(particles, weights, edges):
    n_nodes = particles.shape[0]
    e = edges.shape[1]
    n_macro = e // _CHUNK
    table = particles.reshape(n_nodes, _C)
    edges4 = edges.reshape(2, n_macro, _SUB, _G)
    scratch = (
        [pltpu.VMEM((_SUB, _G), jnp.int32) for _ in range(2 * _NS)]
        + [pltpu.VMEM((_CHUNK, _C), jnp.float32) for _ in range(2 * _NS)]
        + [pltpu.VMEM((_CHUNK, _C), jnp.float32) for _ in range(_NS)]
        + [pltpu.SemaphoreType.DMA for _ in range(3 * _NS)]
    )
    call = pl.kernel(
        _sc_body,
        out_type=jax.ShapeDtypeStruct((e, _C), jnp.float32),
        mesh=plsc.VectorSubcoreMesh(core_axis_name="c", subcore_axis_name="s"),
        compiler_params=pltpu.CompilerParams(
            needs_layout_passes=False, use_tc_tiling_on_sc=False
        ),
        scratch_types=scratch,
    )
    rr = call(edges4, table)
    ww = pl.pallas_call(
        _w_body,
        grid=(e // _WB,),
        in_specs=[pl.BlockSpec((1, _P), lambda i: (0, 0))],
        out_specs=pl.BlockSpec((_WB, _P), lambda i: (i, 0)),
        out_shape=jax.ShapeDtypeStruct((e, _P), jnp.float32),
    )(weights)
    return rr.reshape(e, _P, 4), ww


# speedup vs baseline: 1.0624x; 1.0624x over previous
import jax
import jax.numpy as jnp
from jax import lax
from jax.experimental import pallas as pl
from jax.experimental.pallas import tpu as pltpu
from jax.experimental.pallas import tpu_sc as plsc

_P = 8
_C = 32
_G = 128
_SUB = 2
_CHUNK = _G * _SUB
_NS = 4
_NW = 32
_L = 16


def _sc_body(edges, table, wrow_in, ratios, wout, *refs):
    idx_i = list(refs[0:_NS])
    idx_j = list(refs[_NS:2 * _NS])
    rows_i = list(refs[2 * _NS:3 * _NS])
    rows_j = list(refs[3 * _NS:4 * _NS])
    out = list(refs[4 * _NS:5 * _NS])
    wrow = refs[5 * _NS]
    sem_idx = list(refs[5 * _NS + 1:6 * _NS + 1])
    sem_g = list(refs[6 * _NS + 1:7 * _NS + 1])
    sem_o = list(refs[7 * _NS + 1:8 * _NS + 1])
    sem_w = list(refs[8 * _NS + 1:9 * _NS + 1])

    total = edges.shape[1]
    wid = lax.axis_index("s") * 2 + lax.axis_index("c")
    pltpu.sync_copy(wrow_in, wrow)
    iota = lax.broadcasted_iota(jnp.int32, (_L,), 0)

    kmax = (total + _NW - 1) // _NW
    kmax += (-kmax) % _NS
    nsteps = kmax // _NS

    def fire_idx(b, c):
        pltpu.async_copy(edges.at[0, c], idx_i[b], sem_idx[b])
        pltpu.async_copy(edges.at[1, c], idx_j[b], sem_idx[b])

    def wait_idx(b, c):
        pltpu.make_async_copy(edges.at[0, c], idx_i[b], sem_idx[b]).wait()
        pltpu.make_async_copy(edges.at[1, c], idx_j[b], sem_idx[b]).wait()

    def fire_gathers(b):
        for r in range(_SUB):
            dst_i = rows_i[b].at[pl.ds(r * _G, _G), :]
            dst_j = rows_j[b].at[pl.ds(r * _G, _G), :]
            pltpu.async_copy(table.at[idx_i[b].at[r]], dst_i, sem_g[b])
            pltpu.async_copy(table.at[idx_j[b].at[r]], dst_j, sem_g[b])

    def wait_gathers(b):
        for r in range(_SUB):
            dst_i = rows_i[b].at[pl.ds(r * _G, _G), :]
            dst_j = rows_j[b].at[pl.ds(r * _G, _G), :]
            pltpu.make_async_copy(table.at[idx_i[b].at[r]], dst_i,
                                  sem_g[b]).wait()
            pltpu.make_async_copy(table.at[idx_j[b].at[r]], dst_j,
                                  sem_g[b]).wait()

    def fire_out(b, c):
        pltpu.async_copy(out[b],
                         ratios.at[pl.ds(c * _CHUNK, _CHUNK), :],
                         sem_o[b])
        pltpu.async_copy(wrow, wout.at[pl.ds(c * _CHUNK, _CHUNK), :],
                         sem_w[b])

    def wait_out(b, c):
        pltpu.make_async_copy(out[b],
                              ratios.at[pl.ds(c * _CHUNK, _CHUNK), :],
                              sem_o[b]).wait()
        pltpu.make_async_copy(wrow,
                              wout.at[pl.ds(c * _CHUNK, _CHUNK), :],
                              sem_w[b]).wait()

    def compute(b):
        def g_body(g, carry):
            e0 = g * _L + iota
            for p in range(_P):
                c0 = jnp.full((_L,), 4 * p, jnp.int32)
                c1 = c0 + 1
                c2 = c0 + 2
                c3 = c0 + 3
                pw = plsc.load_gather(rows_i[b], [e0, c0])
                px = plsc.load_gather(rows_i[b], [e0, c1])
                py = plsc.load_gather(rows_i[b], [e0, c2])
                pz = plsc.load_gather(rows_i[b], [e0, c3])
                qw = plsc.load_gather(rows_j[b], [e0, c0])
                qx = plsc.load_gather(rows_j[b], [e0, c1])
                qy = plsc.load_gather(rows_j[b], [e0, c2])
                qz = plsc.load_gather(rows_j[b], [e0, c3])
                r = 1.0 / (qw * qw + qx * qx + qy * qy + qz * qz)
                ow = (pw * qw + px * qx + py * qy + pz * qz) * r
                ox = (px * qw - pw * qx - py * qz + pz * qy) * r
                oy = (py * qw - pw * qy + px * qz - pz * qx) * r
                oz = (pz * qw - pw * qz - px * qy + py * qx) * r
                plsc.store_scatter(out[b], [e0, c0], ow)
                plsc.store_scatter(out[b], [e0, c1], ox)
                plsc.store_scatter(out[b], [e0, c2], oy)
                plsc.store_scatter(out[b], [e0, c3], oz)
            return carry

        lax.fori_loop(0, _CHUNK // _L, g_body, 0)

    for b in range(_NS):
        fire_idx(b, wid + b * _NW)
    for b in range(_NS):
        wait_idx(b, wid + b * _NW)
        fire_gathers(b)

    def step(t, carry):
        for b in range(_NS):
            k = _NS * t + b
            c = wid + k * _NW
            cn = c + _NS * _NW

            def process():
                wait_gathers(b)

                @pl.when(cn < total)
                def _():
                    fire_idx(b, cn)

                @pl.when(t >= 1)
                def _():
                    wait_out(b, c - _NS * _NW)

                compute(b)
                fire_out(b, c)

                @pl.when(cn < total)
                def _():
                    wait_idx(b, cn)
                    fire_gathers(b)

            pl.when(c < total)(process)
        return carry

    lax.fori_loop(0, nsteps, step, 0)

    for b in range(_NS):
        c_last = wid + (kmax - _NS + b) * _NW

        @pl.when(c_last < total)
        def _():
            wait_out(b, c_last)


def kernel(particles, weights, edges):
    n_nodes = particles.shape[0]
    e = edges.shape[1]
    n_macro = e // _CHUNK
    table = particles.reshape(n_nodes, _C)
    edges4 = edges.reshape(2, n_macro, _SUB, _G)
    wrow_in = jnp.tile(weights, (_CHUNK, 1))
    scratch = (
        [pltpu.VMEM((_SUB, _G), jnp.int32) for _ in range(2 * _NS)]
        + [pltpu.VMEM((_CHUNK, _C), jnp.float32) for _ in range(2 * _NS)]
        + [pltpu.VMEM((_CHUNK, _C), jnp.float32) for _ in range(_NS)]
        + [pltpu.VMEM((_CHUNK, _P), jnp.float32)]
        + [pltpu.SemaphoreType.DMA for _ in range(4 * _NS)]
    )
    call = pl.kernel(
        _sc_body,
        out_type=[
            jax.ShapeDtypeStruct((e, _C), jnp.float32),
            jax.ShapeDtypeStruct((e, _P), jnp.float32),
        ],
        mesh=plsc.VectorSubcoreMesh(core_axis_name="c", subcore_axis_name="s"),
        compiler_params=pltpu.CompilerParams(
            needs_layout_passes=False, use_tc_tiling_on_sc=False
        ),
        scratch_types=scratch,
    )
    rr, ww = call(edges4, table, wrow_in)
    return rr.reshape(e, _P, 4), ww
